# baseline (speedup 1.0000x reference)
import jax
import jax.numpy as jnp
from jax.experimental import pallas as pl
from jax.experimental.pallas import tpu as pltpu
from jax.experimental.pallas import tpu_sc as plsc

_W = 128


def kernel(input_ids, embedding_table):
    batch, seq = input_ids.shape
    dim = embedding_table.shape[1]
    n = batch * seq
    ids_flat = input_ids.reshape(1, n).astype(jnp.int32)

    mesh = plsc.VectorSubcoreMesh(core_axis_name="core",
                                  subcore_axis_name="subcore")

    @pl.kernel(out_type=jax.ShapeDtypeStruct((n, dim), embedding_table.dtype),
               mesh=mesh,
               compiler_params=pltpu.CompilerParams(use_tc_tiling_on_sc=False))
    def gather_kernel(table_hbm, i_hbm, o_hbm):
        def body(i_vmem, o_vmem):
            pltpu.sync_copy(table_hbm.at[i_vmem.at[0]], o_vmem)

        pltpu.emit_pipeline(
            body,
            grid=(n // _W,),
            in_specs=[pl.BlockSpec((1, _W), lambda i: (0, i))],
            out_specs=[pl.BlockSpec((_W, dim), lambda i: (i, 0))],
            core_axis_name=("core", "subcore"),
            dimension_semantics=(pltpu.PARALLEL,),
        )(i_hbm, o_hbm)

    out = gather_kernel(embedding_table, ids_flat)
    embeddings = out.reshape(batch, seq, dim)
    attention_mask = jnp.ones((batch, seq), dtype=jnp.int32)
    return (embeddings, attention_mask)


# speedup vs baseline: 1.0274x; 1.0274x over previous
import jax
import jax.numpy as jnp
from jax.experimental import pallas as pl
from jax.experimental.pallas import tpu as pltpu
from jax.experimental.pallas import tpu_sc as plsc

_W = 128
_K = 5


def kernel(input_ids, embedding_table):
    batch, seq = input_ids.shape
    dim = embedding_table.shape[1]
    n = batch * seq
    step_rows = _K * _W
    ids_2d = input_ids.reshape(n // _W, _W).astype(jnp.int32)

    mesh = plsc.VectorSubcoreMesh(core_axis_name="core",
                                  subcore_axis_name="subcore")

    @pl.kernel(out_type=jax.ShapeDtypeStruct((n, dim), embedding_table.dtype),
               mesh=mesh,
               scratch_types=[pltpu.SemaphoreType.DMA],
               compiler_params=pltpu.CompilerParams(use_tc_tiling_on_sc=False))
    def gather_kernel(table_hbm, i_hbm, o_hbm, sem):
        def body(i_vmem, o_vmem):
            copies = [
                pltpu.async_copy(table_hbm.at[i_vmem.at[j]],
                                 o_vmem.at[pl.ds(j * _W, _W), :], sem)
                for j in range(_K)
            ]
            for c in copies:
                c.wait()

        pltpu.emit_pipeline(
            body,
            grid=(n // step_rows,),
            in_specs=[pl.BlockSpec((_K, _W), lambda i: (i, 0))],
            out_specs=[pl.BlockSpec((step_rows, dim), lambda i: (i, 0))],
            core_axis_name=("core", "subcore"),
            dimension_semantics=(pltpu.PARALLEL,),
        )(i_hbm, o_hbm)

    out = gather_kernel(embedding_table, ids_2d)
    embeddings = out.reshape(batch, seq, dim)
    attention_mask = jnp.ones((batch, seq), dtype=jnp.int32)
    return (embeddings, attention_mask)
